# speedup vs baseline: 1.6364x; 1.6364x over previous
import functools

import jax
import jax.numpy as jnp
from jax.experimental import pallas as pl
from jax.experimental.pallas import tpu as pltpu

N_NODES = 10000
ROW_BLOCK = 512
N_PAD = 10240


def _mm_kernel(x_ref, w_ref, o_ref):
    o_ref[...] = jnp.dot(x_ref[...], w_ref[...],
                         preferred_element_type=jnp.float32)


def _linear(x, W):
    n = x.shape[0]
    xp = jnp.pad(x, ((0, N_PAD - n), (0, 0)))
    out = pl.pallas_call(
        _mm_kernel,
        grid=(N_PAD // ROW_BLOCK,),
        in_specs=[
            pl.BlockSpec((ROW_BLOCK, x.shape[1]), lambda i: (i, 0)),
            pl.BlockSpec((W.shape[0], W.shape[1]), lambda i: (0, 0)),
        ],
        out_specs=pl.BlockSpec((ROW_BLOCK, W.shape[1]), lambda i: (i, 0)),
        out_shape=jax.ShapeDtypeStruct((N_PAD, W.shape[1]), jnp.float32),
    )(xp, W)
    return out[:n]


def kernel(x, edge_index, W1, b1, W2, b2, W3, b3):
    n = x.shape[0]
    loops = jnp.arange(n, dtype=edge_index.dtype)
    src = jnp.concatenate([edge_index[0], loops])
    dst = jnp.concatenate([edge_index[1], loops])
    deg = jnp.zeros((n,), dtype=x.dtype).at[dst].add(1.0)
    dinv = jnp.where(deg > 0, jax.lax.rsqrt(deg), 0.0)

    def conv(h, W, b):
        y = _linear(h, W) * dinv[:, None]
        s = jax.ops.segment_sum(jnp.take(y, src, axis=0), dst, num_segments=n)
        return s * dinv[:, None] + b

    h = jax.nn.relu(conv(x, W1, b1))
    h = jax.nn.relu(conv(h, W2, b2))
    return conv(h, W3, b3)


# speedup vs baseline: 4.7379x; 2.8953x over previous
import functools

import jax
import jax.numpy as jnp
from jax import lax
from jax.experimental import pallas as pl
from jax.experimental.pallas import tpu as pltpu
from jax.experimental.pallas import tpu_sc as plsc

N_NODES = 10000
D = 128
NP = 10240
NC, NS = 2, 16
NTILE = NC * NS
BLK = 128
NBLK = 84
EPT = NBLK * BLK
E_PAD = NTILE * EPT
PAD_SRC = 10016
PAD_DST = 10239
ROWS_PER_TILE = NP // NS
SHIFT = 14
MASK = (1 << SHIFT) - 1

_MESH = plsc.VectorSubcoreMesh(
    core_axis_name="c", subcore_axis_name="s", num_cores=NC, num_subcores=NS)

ROW_BLOCK = 512
_TC_GRID = NP // ROW_BLOCK


def _unpack_src(pk_v, j, out_ref):
    for k in range(8):
        v = pk_v[j, 0, pl.ds(16 * k, 16)]
        out_ref[0, pl.ds(16 * k, 16)] = v & MASK


def _unpack_dst(pk_v, j, out_ref):
    for k in range(8):
        v = pk_v[j, 0, pl.ds(16 * k, 16)]
        out_ref[0, pl.ds(16 * k, 16)] = v >> SHIFT


def _zero_rows(buf, nrows, ncols):
    @pl.loop(0, nrows)
    def _z(i):
        for k in range(ncols // 16):
            buf[i, pl.ds(16 * k, 16)] = jnp.zeros((16,), jnp.float32)


def _zero_acc_slice(buf, acc, s):
    @pl.loop(0, ROWS_PER_TILE // BLK)
    def _zc(t):
        pltpu.sync_copy(buf, acc.at[pl.ds(s * ROWS_PER_TILE + t * BLK, BLK)])


def _deg_body(pk_hbm, deg_hbm, pk_v, ones_v, ds_v, acc, sem):
    c = lax.axis_index("c")
    s = lax.axis_index("s")
    wid = c * NS + s

    _zero_rows(ones_v, BLK, D)
    _zero_acc_slice(ones_v, acc, s)

    @pl.loop(0, BLK)
    def _fill(i):
        for k in range(D // 16):
            ones_v[i, pl.ds(16 * k, 16)] = jnp.ones((16,), jnp.float32)

    pltpu.sync_copy(pk_hbm.at[pl.ds(wid * NBLK, NBLK)], pk_v)
    plsc.subcore_barrier()

    @pl.loop(0, NBLK)
    def _scatter(j):
        _unpack_dst(pk_v, j, ds_v)
        pltpu.sync_copy(ones_v, acc.at[ds_v.at[0]], add=True)

    plsc.subcore_barrier()
    pltpu.sync_copy(acc.at[pl.ds(s * ROWS_PER_TILE, ROWS_PER_TILE)],
                    deg_hbm.at[c].at[pl.ds(s * ROWS_PER_TILE, ROWS_PER_TILE)])


def _sc_degree(pk3):
    kern = pl.kernel(
        _deg_body,
        out_type=jax.ShapeDtypeStruct((NC, NP, D), jnp.float32),
        mesh=_MESH,
        scratch_types=[
            pltpu.VMEM((NBLK, 1, BLK), jnp.int32),
            pltpu.VMEM((BLK, D), jnp.float32),
            pltpu.VMEM((1, BLK), jnp.int32),
            pltpu.VMEM_SHARED((NP, D), jnp.float32),
            pltpu.SemaphoreType.DMA,
        ],
    )
    return kern(pk3)


def _segsum_body(y_hbm, pk_hbm, out_hbm,
                 pk_v, ss_v, ds_v, b0, b1, acc, sem0, sem1):
    c = lax.axis_index("c")
    s = lax.axis_index("s")
    wid = c * NS + s

    _zero_rows(b0, BLK, D)
    _zero_acc_slice(b0, acc, s)

    pltpu.sync_copy(pk_hbm.at[pl.ds(wid * NBLK, NBLK)], pk_v)
    plsc.subcore_barrier()

    _unpack_src(pk_v, 0, ss_v.at[0])
    pltpu.async_copy(y_hbm.at[ss_v.at[0, 0]], b0, sem0)

    @pl.loop(0, NBLK, step=2)
    def _edge(j):
        _unpack_src(pk_v, j + 1, ss_v.at[1])
        pltpu.async_copy(y_hbm.at[ss_v.at[1, 0]], b1, sem1)

        pltpu.make_async_copy(y_hbm.at[ss_v.at[0, 0]], b0, sem0).wait()
        _unpack_dst(pk_v, j, ds_v)
        pltpu.sync_copy(b0, acc.at[ds_v.at[0]], add=True)

        @pl.when(j + 2 < NBLK)
        def _pref():
            _unpack_src(pk_v, j + 2, ss_v.at[0])
            pltpu.async_copy(y_hbm.at[ss_v.at[0, 0]], b0, sem0)

        pltpu.make_async_copy(y_hbm.at[ss_v.at[1, 0]], b1, sem1).wait()
        _unpack_dst(pk_v, j + 1, ds_v)
        pltpu.sync_copy(b1, acc.at[ds_v.at[0]], add=True)

    plsc.subcore_barrier()
    pltpu.sync_copy(acc.at[pl.ds(s * ROWS_PER_TILE, ROWS_PER_TILE)],
                    out_hbm.at[c].at[pl.ds(s * ROWS_PER_TILE, ROWS_PER_TILE)])


def _sc_segsum(y, pk3):
    kern = pl.kernel(
        _segsum_body,
        out_type=jax.ShapeDtypeStruct((NC, NP, D), jnp.float32),
        mesh=_MESH,
        scratch_types=[
            pltpu.VMEM((NBLK, 1, BLK), jnp.int32),
            pltpu.VMEM((2, 1, BLK), jnp.int32),
            pltpu.VMEM((1, BLK), jnp.int32),
            pltpu.VMEM((BLK, D), jnp.float32),
            pltpu.VMEM((BLK, D), jnp.float32),
            pltpu.VMEM_SHARED((NP, D), jnp.float32),
            pltpu.SemaphoreType.DMA,
            pltpu.SemaphoreType.DMA,
        ],
    )
    return kern(y, pk3)


def _l1_body(deg_ref, x_ref, w_ref, y_ref, dinv_ref):
    deg = deg_ref[...]
    dinv = jnp.where(deg > 0, lax.rsqrt(deg), 0.0)
    dinv_ref[...] = dinv
    y_ref[...] = jnp.dot(x_ref[...], w_ref[...],
                         preferred_element_type=jnp.float32) * dinv


def _tc_layer1(deg_col, x, W):
    return pl.pallas_call(
        _l1_body,
        grid=(_TC_GRID,),
        in_specs=[
            pl.BlockSpec((ROW_BLOCK, 1), lambda i: (i, 0)),
            pl.BlockSpec((ROW_BLOCK, D), lambda i: (i, 0)),
            pl.BlockSpec((D, D), lambda i: (0, 0)),
        ],
        out_specs=[
            pl.BlockSpec((ROW_BLOCK, D), lambda i: (i, 0)),
            pl.BlockSpec((ROW_BLOCK, 1), lambda i: (i, 0)),
        ],
        out_shape=[
            jax.ShapeDtypeStruct((NP, D), jnp.float32),
            jax.ShapeDtypeStruct((NP, 1), jnp.float32),
        ],
    )(deg_col, x, W)


def _fused_body(p_ref, dinv_ref, b_ref, w_ref, y_ref):
    dinv = dinv_ref[...]
    h = jnp.maximum(dinv * (p_ref[0] + p_ref[1]) + b_ref[...], 0.0)
    y_ref[...] = jnp.dot(h, w_ref[...],
                         preferred_element_type=jnp.float32) * dinv


def _tc_fused(parts, dinv_col, b, W):
    return pl.pallas_call(
        _fused_body,
        grid=(_TC_GRID,),
        in_specs=[
            pl.BlockSpec((NC, ROW_BLOCK, D), lambda i: (0, i, 0)),
            pl.BlockSpec((ROW_BLOCK, 1), lambda i: (i, 0)),
            pl.BlockSpec((1, D), lambda i: (0, 0)),
            pl.BlockSpec((D, D), lambda i: (0, 0)),
        ],
        out_specs=pl.BlockSpec((ROW_BLOCK, D), lambda i: (i, 0)),
        out_shape=jax.ShapeDtypeStruct((NP, D), jnp.float32),
    )(parts, dinv_col, b.reshape(1, D), W)


def _epi_body(p_ref, dinv_ref, b_ref, o_ref):
    o_ref[...] = dinv_ref[...] * (p_ref[0] + p_ref[1]) + b_ref[...]


def _tc_epilogue(parts, dinv_col, b):
    return pl.pallas_call(
        _epi_body,
        grid=(_TC_GRID,),
        in_specs=[
            pl.BlockSpec((NC, ROW_BLOCK, D), lambda i: (0, i, 0)),
            pl.BlockSpec((ROW_BLOCK, 1), lambda i: (i, 0)),
            pl.BlockSpec((1, D), lambda i: (0, 0)),
        ],
        out_specs=pl.BlockSpec((ROW_BLOCK, D), lambda i: (i, 0)),
        out_shape=jax.ShapeDtypeStruct((NP, D), jnp.float32),
    )(parts, dinv_col, b.reshape(1, D))


def kernel(x, edge_index, W1, b1, W2, b2, W3, b3):
    n = x.shape[0]
    loops = jnp.arange(n, dtype=edge_index.dtype)
    n_real = edge_index.shape[1] + n
    pad = E_PAD - n_real
    src = jnp.concatenate(
        [edge_index[0], loops, jnp.full((pad,), PAD_SRC, edge_index.dtype)])
    dst = jnp.concatenate(
        [edge_index[1], loops, jnp.full((pad,), PAD_DST, edge_index.dtype)])
    packed = src | (dst << SHIFT)
    pk3 = packed.reshape(E_PAD // BLK, 1, BLK)

    xp = jnp.pad(x, ((0, NP - n), (0, 0)))

    degp = _sc_degree(pk3)
    deg_col = (degp[0] + degp[1])[:, :1]

    y1, dinv_col = _tc_layer1(deg_col, xp, W1)
    s1 = _sc_segsum(y1, pk3)
    y2 = _tc_fused(s1, dinv_col, b1, W2)
    s2 = _sc_segsum(y2, pk3)
    y3 = _tc_fused(s2, dinv_col, b2, W3)
    s3 = _sc_segsum(y3, pk3)
    out = _tc_epilogue(s3, dinv_col, b3)
    return out[:n]
